# baseline (speedup 1.0000x reference)
import functools

import jax
import jax.numpy as jnp
from jax import lax
from jax.experimental import pallas as pl
from jax.experimental.pallas import tpu as pltpu
from jax.experimental.pallas import tpu_sc as plsc

F32 = jnp.float32
I32 = jnp.int32

NC = 2
NS = 16
LANES = 16
NW = NC * NS


def _mesh():
    return plsc.VectorSubcoreMesh(core_axis_name="c", subcore_axis_name="s")


_SC_PARAMS = pltpu.CompilerParams(needs_layout_passes=False,
                                  use_tc_tiling_on_sc=False)


def _lane_iota():
    return lax.iota(I32, LANES)


def _bcast(w16, j):
    return jnp.take_along_axis(w16, jnp.full((LANES,), j, I32), axis=0)


def _make_deg(EP, NP):
    EW = EP // NW
    K = 128
    NCH = EW // K
    NPAIR = NCH // 2
    GPC = K // LANES
    ZR = NP // NS

    @functools.partial(
        pl.kernel,
        mesh=_mesh(),
        out_type=jax.ShapeDtypeStruct((NC, NP, LANES), F32),
        compiler_params=_SC_PARAMS,
        scratch_types=[
            pltpu.VMEM_SHARED((NP, LANES), F32),
            pltpu.VMEM((ZR, LANES), F32),
            pltpu.VMEM((K,), I32), pltpu.VMEM((K,), F32),
            pltpu.VMEM((K,), I32), pltpu.VMEM((K, LANES), F32),
            pltpu.VMEM((K,), I32), pltpu.VMEM((K,), F32),
            pltpu.VMEM((K,), I32), pltpu.VMEM((K, LANES), F32),
            pltpu.SemaphoreType.DMA, pltpu.SemaphoreType.DMA,
            pltpu.SemaphoreType.DMA, pltpu.SemaphoreType.DMA,
        ],
    )
    def deg_kernel(col_hbm, ea_hbm, out_hbm, acc, zbuf,
                   col_a, ea_a, scol_a, rep_a,
                   col_b, ea_b, scol_b, rep_b,
                   si_a, si_b, ss_a, ss_b):
        c = lax.axis_index("c")
        s = lax.axis_index("s")
        zero16 = jnp.zeros((LANES,), F32)
        base0 = (c * NS + s) * EW

        def issue_idx(i, col_v, ea_v, sem):
            base = base0 + i * K
            pltpu.async_copy(col_hbm.at[pl.ds(base, K)], col_v, sem)
            pltpu.async_copy(ea_hbm.at[pl.ds(base, K)], ea_v, sem)

        def wait_idx(col_v, ea_v, sem):
            pltpu.make_async_copy(col_hbm.at[pl.ds(0, K)], col_v, sem).wait()
            pltpu.make_async_copy(ea_hbm.at[pl.ds(0, K)], ea_v, sem).wait()

        def build_rep(ea_v, rep_v):
            def gbody(g, cc):
                w16 = jnp.maximum(ea_v[pl.ds(g * LANES, LANES)], 0.0)
                for j in range(LANES):
                    rep_v[g * LANES + j] = _bcast(w16, j)
                return cc

            lax.fori_loop(0, GPC, gbody, 0)

        def copy_col(col_v, scol_v):
            for q in range(GPC):
                sl = pl.ds(q * LANES, LANES)
                scol_v[sl] = col_v[sl]

        issue_idx(0, col_a, ea_a, si_a)
        issue_idx(1, col_b, ea_b, si_b)

        def zbody(i, carry):
            zbuf[i] = zero16
            return carry

        lax.fori_loop(0, ZR, zbody, 0)
        pltpu.sync_copy(zbuf, acc.at[pl.ds(ZR * s, ZR)])
        plsc.subcore_barrier()

        def pair(p, carry):
            wait_idx(col_a, ea_a, si_a)

            @pl.when(p > 0)
            def _():
                pltpu.make_async_copy(rep_a, acc.at[scol_a], ss_a).wait()

            copy_col(col_a, scol_a)
            build_rep(ea_a, rep_a)
            pltpu.async_copy(rep_a, acc.at[scol_a], ss_a, add=True)

            @pl.when(p < NPAIR - 1)
            def _():
                issue_idx(2 * p + 2, col_a, ea_a, si_a)

            wait_idx(col_b, ea_b, si_b)

            @pl.when(p > 0)
            def _():
                pltpu.make_async_copy(rep_b, acc.at[scol_b], ss_b).wait()

            copy_col(col_b, scol_b)
            build_rep(ea_b, rep_b)
            pltpu.async_copy(rep_b, acc.at[scol_b], ss_b, add=True)

            @pl.when(p < NPAIR - 1)
            def _():
                issue_idx(2 * p + 3, col_b, ea_b, si_b)

            return carry

        lax.fori_loop(0, NPAIR, pair, 0)
        pltpu.make_async_copy(rep_a, acc.at[scol_a], ss_a).wait()
        pltpu.make_async_copy(rep_b, acc.at[scol_b], ss_b).wait()
        plsc.subcore_barrier()
        pltpu.sync_copy(acc.at[pl.ds(ZR * s, ZR)],
                        out_hbm.at[c, pl.ds(ZR * s, ZR)])

    return deg_kernel


def _make_agg(EP, NP, Dh):
    EW = EP // NS
    K = 80
    NCH = EW // K
    RW = NP // NS
    JG = Dh // LANES

    NPAIR = NCH // 2
    GPC = K // LANES

    @functools.partial(
        pl.kernel,
        mesh=_mesh(),
        out_type=(jax.ShapeDtypeStruct((NP, Dh), F32),
                  jax.ShapeDtypeStruct((NP, Dh), F32)),
        compiler_params=_SC_PARAMS,
        scratch_types=[
            pltpu.VMEM_SHARED((NP, Dh), F32),
            pltpu.VMEM((K,), I32), pltpu.VMEM((K,), I32),
            pltpu.VMEM((K,), F32), pltpu.VMEM((K,), I32),
            pltpu.VMEM((K, Dh), F32),
            pltpu.VMEM((K,), I32), pltpu.VMEM((K,), I32),
            pltpu.VMEM((K,), F32), pltpu.VMEM((K,), I32),
            pltpu.VMEM((K, Dh), F32),
            pltpu.SemaphoreType.DMA, pltpu.SemaphoreType.DMA,
            pltpu.SemaphoreType.DMA, pltpu.SemaphoreType.DMA,
            pltpu.SemaphoreType.DMA, pltpu.SemaphoreType.DMA,
        ],
    )
    def agg_kernel(row_hbm, col_hbm, ea_hbm, xq_lo, xq_hi, init_lo, init_hi,
                   out_lo, out_hi, acc,
                   row_a, col_a, ea_a, scol_a, rows_a,
                   row_b, col_b, ea_b, scol_b, rows_b,
                   si_a, si_b, sg_a, sg_b, ss_a, ss_b):
        c = lax.axis_index("c")
        s = lax.axis_index("s")
        lane = _lane_iota()

        def body(xq_h, init_h, out_h):
            base0 = s * EW

            def issue_idx(i, row_v, col_v, ea_v, sem):
                base = base0 + i * K
                pltpu.async_copy(row_hbm.at[pl.ds(base, K)], row_v, sem)
                pltpu.async_copy(col_hbm.at[pl.ds(base, K)], col_v, sem)
                pltpu.async_copy(ea_hbm.at[pl.ds(base, K)], ea_v, sem)

            def wait_idx(row_v, col_v, ea_v, sem):
                pltpu.make_async_copy(row_hbm.at[pl.ds(0, K)], row_v,
                                      sem).wait()
                pltpu.make_async_copy(col_hbm.at[pl.ds(0, K)], col_v,
                                      sem).wait()
                pltpu.make_async_copy(ea_hbm.at[pl.ds(0, K)], ea_v,
                                      sem).wait()

            def wait_scatter(rows_v, scol_v, sem):
                pltpu.make_async_copy(rows_v, acc.at[scol_v], sem).wait()

            def scale(ea_v, rows_v):
                def gbody(g, cc):
                    w16 = jnp.maximum(ea_v[pl.ds(g * LANES, LANES)], 0.0)
                    for j in range(LANES):
                        nb = _bcast(w16, j)
                        e = g * LANES + j
                        for jj in range(JG):
                            sl = pl.ds(jj * LANES, LANES)
                            rows_v[e, sl] = rows_v[e, sl] * nb
                    return cc

                lax.fori_loop(0, GPC, gbody, 0)

            def copy_col(col_v, scol_v):
                for q in range(GPC):
                    sl = pl.ds(q * LANES, LANES)
                    scol_v[sl] = col_v[sl]

            issue_idx(0, row_a, col_a, ea_a, si_a)
            issue_idx(1, row_b, col_b, ea_b, si_b)
            pltpu.sync_copy(init_h.at[pl.ds(RW * s, RW)],
                            acc.at[pl.ds(RW * s, RW)])
            plsc.subcore_barrier()

            def pair(p, carry):
                wait_idx(row_a, col_a, ea_a, si_a)

                @pl.when(p > 0)
                def _():
                    wait_scatter(rows_a, scol_a, ss_a)

                pltpu.async_copy(xq_h.at[row_a], rows_a, sg_a)

                @pl.when(p > 0)
                def _():
                    wait_scatter(rows_b, scol_b, ss_b)

                wait_idx(row_b, col_b, ea_b, si_b)
                pltpu.async_copy(xq_h.at[row_b], rows_b, sg_b)

                pltpu.make_async_copy(xq_h.at[row_a], rows_a, sg_a).wait()
                copy_col(col_a, scol_a)
                scale(ea_a, rows_a)
                pltpu.async_copy(rows_a, acc.at[scol_a], ss_a, add=True)

                @pl.when(p < NPAIR - 1)
                def _():
                    issue_idx(2 * p + 2, row_a, col_a, ea_a, si_a)

                pltpu.make_async_copy(xq_h.at[row_b], rows_b, sg_b).wait()
                copy_col(col_b, scol_b)
                scale(ea_b, rows_b)
                pltpu.async_copy(rows_b, acc.at[scol_b], ss_b, add=True)

                @pl.when(p < NPAIR - 1)
                def _():
                    issue_idx(2 * p + 3, row_b, col_b, ea_b, si_b)

                return carry

            lax.fori_loop(0, NPAIR, pair, 0)
            wait_scatter(rows_a, scol_a, ss_a)
            wait_scatter(rows_b, scol_b, ss_b)
            plsc.subcore_barrier()
            pltpu.sync_copy(acc.at[pl.ds(RW * s, RW)],
                            out_h.at[pl.ds(RW * s, RW)])

        @pl.when(c == 0)
        def _():
            body(xq_lo, init_lo, out_lo)

        @pl.when(c == 1)
        def _():
            body(xq_hi, init_hi, out_hi)

    return agg_kernel


def _mm_first(x, W, b, p0, p1, br=2048):
    N, DI = x.shape
    DO = W.shape[1]
    Dh = DO // 2

    def body(x_ref, w_ref, b_ref, p0_ref, p1_ref, xlo, xhi, ilo, ihi):
        deg = p0_ref[...] + p1_ref[...] + 1.0
        di = lax.rsqrt(deg)
        rd = deg * di
        xw = jnp.dot(x_ref[...], w_ref[...], preferred_element_type=F32)
        xq = xw * di
        init = xq + b_ref[...] * rd
        xlo[...] = xq[:, :Dh]
        xhi[...] = xq[:, Dh:]
        ilo[...] = init[:, :Dh]
        ihi[...] = init[:, Dh:]

    outs = tuple(jax.ShapeDtypeStruct((N, Dh), F32) for _ in range(4))
    bo = pl.BlockSpec((br, Dh), lambda i: (i, 0))
    bc = pl.BlockSpec((br, 1), lambda i: (i, 0))
    return pl.pallas_call(
        body,
        grid=(N // br,),
        in_specs=[pl.BlockSpec((br, DI), lambda i: (i, 0)),
                  pl.BlockSpec((DI, DO), lambda i: (0, 0)),
                  pl.BlockSpec((1, DO), lambda i: (0, 0)),
                  bc, bc],
        out_specs=[bo, bo, bo, bo],
        out_shape=outs,
    )(x, W, b.reshape(1, DO), p0, p1)


def _mm_mid(slo, shi, W, b, p0, p1, br=2048):
    N, Dhin = slo.shape
    DI, DO = W.shape
    Dh = DO // 2

    def body(lo_ref, hi_ref, w_ref, b_ref, p0_ref, p1_ref,
             xlo, xhi, ilo, ihi):
        deg = p0_ref[...] + p1_ref[...] + 1.0
        di = lax.rsqrt(deg)
        rd = deg * di
        hlo = jnp.maximum(lo_ref[...] * di, 0.0)
        hhi = jnp.maximum(hi_ref[...] * di, 0.0)
        w = w_ref[...]
        xw = (jnp.dot(hlo, w[:Dhin], preferred_element_type=F32)
              + jnp.dot(hhi, w[Dhin:], preferred_element_type=F32))
        xq = xw * di
        init = xq + b_ref[...] * rd
        xlo[...] = xq[:, :Dh]
        xhi[...] = xq[:, Dh:]
        ilo[...] = init[:, :Dh]
        ihi[...] = init[:, Dh:]

    outs = tuple(jax.ShapeDtypeStruct((N, Dh), F32) for _ in range(4))
    bi = pl.BlockSpec((br, Dhin), lambda i: (i, 0))
    bo = pl.BlockSpec((br, Dh), lambda i: (i, 0))
    bc = pl.BlockSpec((br, 1), lambda i: (i, 0))
    return pl.pallas_call(
        body,
        grid=(N // br,),
        in_specs=[bi, bi,
                  pl.BlockSpec((DI, DO), lambda i: (0, 0)),
                  pl.BlockSpec((1, DO), lambda i: (0, 0)),
                  bc, bc],
        out_specs=[bo, bo, bo, bo],
        out_shape=outs,
    )(slo, shi, W, b.reshape(1, DO), p0, p1)


def _head(slo, shi, W, b, p0, p1, br=2048):
    N, Dhin = slo.shape
    DI, DO = W.shape

    def body(lo_ref, hi_ref, w_ref, b_ref, p0_ref, p1_ref, o_ref):
        deg = p0_ref[...] + p1_ref[...] + 1.0
        di = lax.rsqrt(deg)
        hlo = jnp.maximum(lo_ref[...] * di, 0.0)
        hhi = jnp.maximum(hi_ref[...] * di, 0.0)
        w = w_ref[...]
        out = (jnp.dot(hlo, w[:Dhin], preferred_element_type=F32)
               + jnp.dot(hhi, w[Dhin:], preferred_element_type=F32))
        out = out + b_ref[...]
        o_ref[...] = jnp.where(out > 0, out, 0.2 * out)

    bi = pl.BlockSpec((br, Dhin), lambda i: (i, 0))
    bc = pl.BlockSpec((br, 1), lambda i: (i, 0))
    return pl.pallas_call(
        body,
        grid=(N // br,),
        in_specs=[bi, bi,
                  pl.BlockSpec((DI, DO), lambda i: (0, 0)),
                  pl.BlockSpec((1, DO), lambda i: (0, 0)),
                  bc, bc],
        out_specs=pl.BlockSpec((br, DO), lambda i: (i, 0)),
        out_shape=jax.ShapeDtypeStruct((N, DO), F32),
    )(slo, shi, W, b.reshape(1, DO), p0, p1)


def kernel(x, edge_index, edge_attr, W1, b1, W2, b2, W3, b3, W_fc3, b_fc3):
    N, DI = x.shape
    E = edge_index.shape[1]
    NP = ((N + 2047) // 2048) * 2048

    EP = ((E + NW * 256 - 1) // (NW * 256)) * (NW * 256)
    row = jnp.pad(edge_index[0], (0, EP - E))
    col = jnp.pad(edge_index[1], (0, EP - E))
    ea = jnp.pad(edge_attr, (0, EP - E))
    xp = jnp.pad(x, ((0, NP - N), (0, 0)))

    deg_part = _make_deg(EP, NP)(col, ea)
    p0 = deg_part[0, :, 0:1]
    p1 = deg_part[1, :, 0:1]

    xq_lo, xq_hi, i_lo, i_hi = _mm_first(xp, W1, b1, p0, p1)
    s_lo, s_hi = _make_agg(EP, NP, W1.shape[1] // 2)(
        row, col, ea, xq_lo, xq_hi, i_lo, i_hi)

    xq_lo, xq_hi, i_lo, i_hi = _mm_mid(s_lo, s_hi, W2, b2, p0, p1)
    s_lo, s_hi = _make_agg(EP, NP, W2.shape[1] // 2)(
        row, col, ea, xq_lo, xq_hi, i_lo, i_hi)

    xq_lo, xq_hi, i_lo, i_hi = _mm_mid(s_lo, s_hi, W3, b3, p0, p1)
    s_lo, s_hi = _make_agg(EP, NP, W3.shape[1] // 2)(
        row, col, ea, xq_lo, xq_hi, i_lo, i_hi)

    return _head(s_lo, s_hi, W_fc3, b_fc3, p0, p1)[:N]


# speedup vs baseline: 1.0966x; 1.0966x over previous
import functools

import jax
import jax.numpy as jnp
from jax import lax
from jax.experimental import pallas as pl
from jax.experimental.pallas import tpu as pltpu
from jax.experimental.pallas import tpu_sc as plsc

F32 = jnp.float32
I32 = jnp.int32

NC = 2
NS = 16
LANES = 16
NW = NC * NS


def _mesh():
    return plsc.VectorSubcoreMesh(core_axis_name="c", subcore_axis_name="s")


_SC_PARAMS = pltpu.CompilerParams(needs_layout_passes=False,
                                  use_tc_tiling_on_sc=False)


def _lane_iota():
    return lax.iota(I32, LANES)


def _bcast(w16, j):
    return jnp.take_along_axis(w16, jnp.full((LANES,), j, I32), axis=0)


def _make_deg(EP, NP):
    EW = EP // NW
    K = 128
    NCH = EW // K
    NPAIR = NCH // 2
    GPC = K // LANES
    ZR = NP // NS

    @functools.partial(
        pl.kernel,
        mesh=_mesh(),
        out_type=jax.ShapeDtypeStruct((NC, NP, LANES), F32),
        compiler_params=_SC_PARAMS,
        scratch_types=[
            pltpu.VMEM_SHARED((NP, LANES), F32),
            pltpu.VMEM((ZR, LANES), F32),
            pltpu.VMEM((K,), I32), pltpu.VMEM((K,), F32),
            pltpu.VMEM((K,), I32), pltpu.VMEM((K, LANES), F32),
            pltpu.VMEM((K,), I32), pltpu.VMEM((K,), F32),
            pltpu.VMEM((K,), I32), pltpu.VMEM((K, LANES), F32),
            pltpu.SemaphoreType.DMA, pltpu.SemaphoreType.DMA,
            pltpu.SemaphoreType.DMA, pltpu.SemaphoreType.DMA,
        ],
    )
    def deg_kernel(col_hbm, ea_hbm, out_hbm, acc, zbuf,
                   col_a, ea_a, scol_a, rep_a,
                   col_b, ea_b, scol_b, rep_b,
                   si_a, si_b, ss_a, ss_b):
        c = lax.axis_index("c")
        s = lax.axis_index("s")
        zero16 = jnp.zeros((LANES,), F32)
        base0 = (c * NS + s) * EW

        def issue_idx(i, col_v, ea_v, sem):
            base = base0 + i * K
            pltpu.async_copy(col_hbm.at[pl.ds(base, K)], col_v, sem)
            pltpu.async_copy(ea_hbm.at[pl.ds(base, K)], ea_v, sem)

        def wait_idx(col_v, ea_v, sem):
            pltpu.make_async_copy(col_hbm.at[pl.ds(0, K)], col_v, sem).wait()
            pltpu.make_async_copy(ea_hbm.at[pl.ds(0, K)], ea_v, sem).wait()

        def build_rep(ea_v, rep_v):
            def gbody(g, cc):
                w16 = jnp.maximum(ea_v[pl.ds(g * LANES, LANES)], 0.0)
                for j in range(LANES):
                    rep_v[g * LANES + j] = _bcast(w16, j)
                return cc

            lax.fori_loop(0, GPC, gbody, 0)

        def copy_col(col_v, scol_v):
            for q in range(GPC):
                sl = pl.ds(q * LANES, LANES)
                scol_v[sl] = col_v[sl]

        issue_idx(0, col_a, ea_a, si_a)
        issue_idx(1, col_b, ea_b, si_b)

        def zbody(i, carry):
            zbuf[i] = zero16
            return carry

        lax.fori_loop(0, ZR, zbody, 0)
        pltpu.sync_copy(zbuf, acc.at[pl.ds(ZR * s, ZR)])
        plsc.subcore_barrier()

        def pair(p, carry):
            wait_idx(col_a, ea_a, si_a)

            @pl.when(p > 0)
            def _():
                pltpu.make_async_copy(rep_a, acc.at[scol_a], ss_a).wait()

            copy_col(col_a, scol_a)
            build_rep(ea_a, rep_a)
            pltpu.async_copy(rep_a, acc.at[scol_a], ss_a, add=True)

            @pl.when(p < NPAIR - 1)
            def _():
                issue_idx(2 * p + 2, col_a, ea_a, si_a)

            wait_idx(col_b, ea_b, si_b)

            @pl.when(p > 0)
            def _():
                pltpu.make_async_copy(rep_b, acc.at[scol_b], ss_b).wait()

            copy_col(col_b, scol_b)
            build_rep(ea_b, rep_b)
            pltpu.async_copy(rep_b, acc.at[scol_b], ss_b, add=True)

            @pl.when(p < NPAIR - 1)
            def _():
                issue_idx(2 * p + 3, col_b, ea_b, si_b)

            return carry

        lax.fori_loop(0, NPAIR, pair, 0)
        pltpu.make_async_copy(rep_a, acc.at[scol_a], ss_a).wait()
        pltpu.make_async_copy(rep_b, acc.at[scol_b], ss_b).wait()
        plsc.subcore_barrier()
        pltpu.sync_copy(acc.at[pl.ds(ZR * s, ZR)],
                        out_hbm.at[c, pl.ds(ZR * s, ZR)])

    return deg_kernel


def _make_agg(EP, NP, Dh):
    EW = EP // NS
    K = 80
    NCH = EW // K
    RW = NP // NS
    JG = Dh // LANES

    NPAIR = NCH // 2
    GPC = K // LANES

    @functools.partial(
        pl.kernel,
        mesh=_mesh(),
        out_type=(jax.ShapeDtypeStruct((NP, Dh), F32),
                  jax.ShapeDtypeStruct((NP, Dh), F32)),
        compiler_params=_SC_PARAMS,
        scratch_types=[
            pltpu.VMEM_SHARED((NP, Dh), F32),
            pltpu.VMEM((K,), I32), pltpu.VMEM((K,), I32),
            pltpu.VMEM((K,), F32), pltpu.VMEM((K,), I32),
            pltpu.VMEM((K, Dh), F32),
            pltpu.VMEM((K,), I32), pltpu.VMEM((K,), I32),
            pltpu.VMEM((K,), F32), pltpu.VMEM((K,), I32),
            pltpu.VMEM((K, Dh), F32),
            pltpu.SemaphoreType.DMA, pltpu.SemaphoreType.DMA,
            pltpu.SemaphoreType.DMA, pltpu.SemaphoreType.DMA,
            pltpu.SemaphoreType.DMA, pltpu.SemaphoreType.DMA,
        ],
    )
    def agg_kernel(row_hbm, col_hbm, ea_hbm, xq_lo, xq_hi, init_lo, init_hi,
                   out_lo, out_hi, acc,
                   row_a, col_a, ea_a, scol_a, rows_a,
                   row_b, col_b, ea_b, scol_b, rows_b,
                   si_a, si_b, sg_a, sg_b, ss_a, ss_b):
        c = lax.axis_index("c")
        s = lax.axis_index("s")
        lane = _lane_iota()

        def body(xq_h, init_h, out_h):
            base0 = s * EW

            def issue_idx(i, row_v, col_v, ea_v, sem):
                base = base0 + i * K
                pltpu.async_copy(row_hbm.at[pl.ds(base, K)], row_v, sem)
                pltpu.async_copy(col_hbm.at[pl.ds(base, K)], col_v, sem)
                pltpu.async_copy(ea_hbm.at[pl.ds(base, K)], ea_v, sem)

            def wait_idx(row_v, col_v, ea_v, sem):
                pltpu.make_async_copy(row_hbm.at[pl.ds(0, K)], row_v,
                                      sem).wait()
                pltpu.make_async_copy(col_hbm.at[pl.ds(0, K)], col_v,
                                      sem).wait()
                pltpu.make_async_copy(ea_hbm.at[pl.ds(0, K)], ea_v,
                                      sem).wait()

            def wait_scatter(rows_v, scol_v, sem):
                pltpu.make_async_copy(rows_v, acc.at[scol_v], sem).wait()

            def scale(ea_v, rows_v):
                def gbody(g, cc):
                    w16 = jnp.maximum(ea_v[pl.ds(g * LANES, LANES)], 0.0)
                    for j in range(LANES):
                        nb = jnp.sum(jnp.where(lane == j, w16, 0.0))
                        e = g * LANES + j
                        for jj in range(JG):
                            sl = pl.ds(jj * LANES, LANES)
                            rows_v[e, sl] = rows_v[e, sl] * nb
                    return cc

                lax.fori_loop(0, GPC, gbody, 0)

            def copy_col(col_v, scol_v):
                for q in range(GPC):
                    sl = pl.ds(q * LANES, LANES)
                    scol_v[sl] = col_v[sl]

            issue_idx(0, row_a, col_a, ea_a, si_a)
            issue_idx(1, row_b, col_b, ea_b, si_b)
            pltpu.sync_copy(init_h.at[pl.ds(RW * s, RW)],
                            acc.at[pl.ds(RW * s, RW)])
            plsc.subcore_barrier()

            def pair(p, carry):
                wait_idx(row_a, col_a, ea_a, si_a)

                @pl.when(p > 0)
                def _():
                    wait_scatter(rows_a, scol_a, ss_a)

                pltpu.async_copy(xq_h.at[row_a], rows_a, sg_a)

                @pl.when(p > 0)
                def _():
                    wait_scatter(rows_b, scol_b, ss_b)

                wait_idx(row_b, col_b, ea_b, si_b)
                pltpu.async_copy(xq_h.at[row_b], rows_b, sg_b)

                pltpu.make_async_copy(xq_h.at[row_a], rows_a, sg_a).wait()
                copy_col(col_a, scol_a)
                scale(ea_a, rows_a)
                pltpu.async_copy(rows_a, acc.at[scol_a], ss_a, add=True)

                @pl.when(p < NPAIR - 1)
                def _():
                    issue_idx(2 * p + 2, row_a, col_a, ea_a, si_a)

                pltpu.make_async_copy(xq_h.at[row_b], rows_b, sg_b).wait()
                copy_col(col_b, scol_b)
                scale(ea_b, rows_b)
                pltpu.async_copy(rows_b, acc.at[scol_b], ss_b, add=True)

                @pl.when(p < NPAIR - 1)
                def _():
                    issue_idx(2 * p + 3, row_b, col_b, ea_b, si_b)

                return carry

            lax.fori_loop(0, NPAIR, pair, 0)
            wait_scatter(rows_a, scol_a, ss_a)
            wait_scatter(rows_b, scol_b, ss_b)
            plsc.subcore_barrier()
            pltpu.sync_copy(acc.at[pl.ds(RW * s, RW)],
                            out_h.at[pl.ds(RW * s, RW)])

        @pl.when(c == 0)
        def _():
            body(xq_lo, init_lo, out_lo)

        @pl.when(c == 1)
        def _():
            body(xq_hi, init_hi, out_hi)

    return agg_kernel


def _mm_first(x, W, b, p0, p1, br=2048):
    N, DI = x.shape
    DO = W.shape[1]
    Dh = DO // 2

    def body(x_ref, w_ref, b_ref, p0_ref, p1_ref, xlo, xhi, ilo, ihi):
        deg = p0_ref[...] + p1_ref[...] + 1.0
        di = lax.rsqrt(deg)
        rd = deg * di
        xw = jnp.dot(x_ref[...], w_ref[...], preferred_element_type=F32)
        xq = xw * di
        init = xq + b_ref[...] * rd
        xlo[...] = xq[:, :Dh]
        xhi[...] = xq[:, Dh:]
        ilo[...] = init[:, :Dh]
        ihi[...] = init[:, Dh:]

    outs = tuple(jax.ShapeDtypeStruct((N, Dh), F32) for _ in range(4))
    bo = pl.BlockSpec((br, Dh), lambda i: (i, 0))
    bc = pl.BlockSpec((br, 1), lambda i: (i, 0))
    return pl.pallas_call(
        body,
        grid=(N // br,),
        in_specs=[pl.BlockSpec((br, DI), lambda i: (i, 0)),
                  pl.BlockSpec((DI, DO), lambda i: (0, 0)),
                  pl.BlockSpec((1, DO), lambda i: (0, 0)),
                  bc, bc],
        out_specs=[bo, bo, bo, bo],
        out_shape=outs,
    )(x, W, b.reshape(1, DO), p0, p1)


def _mm_mid(slo, shi, W, b, p0, p1, br=2048):
    N, Dhin = slo.shape
    DI, DO = W.shape
    Dh = DO // 2

    def body(lo_ref, hi_ref, w_ref, b_ref, p0_ref, p1_ref,
             xlo, xhi, ilo, ihi):
        deg = p0_ref[...] + p1_ref[...] + 1.0
        di = lax.rsqrt(deg)
        rd = deg * di
        hlo = jnp.maximum(lo_ref[...] * di, 0.0)
        hhi = jnp.maximum(hi_ref[...] * di, 0.0)
        w = w_ref[...]
        xw = (jnp.dot(hlo, w[:Dhin], preferred_element_type=F32)
              + jnp.dot(hhi, w[Dhin:], preferred_element_type=F32))
        xq = xw * di
        init = xq + b_ref[...] * rd
        xlo[...] = xq[:, :Dh]
        xhi[...] = xq[:, Dh:]
        ilo[...] = init[:, :Dh]
        ihi[...] = init[:, Dh:]

    outs = tuple(jax.ShapeDtypeStruct((N, Dh), F32) for _ in range(4))
    bi = pl.BlockSpec((br, Dhin), lambda i: (i, 0))
    bo = pl.BlockSpec((br, Dh), lambda i: (i, 0))
    bc = pl.BlockSpec((br, 1), lambda i: (i, 0))
    return pl.pallas_call(
        body,
        grid=(N // br,),
        in_specs=[bi, bi,
                  pl.BlockSpec((DI, DO), lambda i: (0, 0)),
                  pl.BlockSpec((1, DO), lambda i: (0, 0)),
                  bc, bc],
        out_specs=[bo, bo, bo, bo],
        out_shape=outs,
    )(slo, shi, W, b.reshape(1, DO), p0, p1)


def _head(slo, shi, W, b, p0, p1, br=2048):
    N, Dhin = slo.shape
    DI, DO = W.shape

    def body(lo_ref, hi_ref, w_ref, b_ref, p0_ref, p1_ref, o_ref):
        deg = p0_ref[...] + p1_ref[...] + 1.0
        di = lax.rsqrt(deg)
        hlo = jnp.maximum(lo_ref[...] * di, 0.0)
        hhi = jnp.maximum(hi_ref[...] * di, 0.0)
        w = w_ref[...]
        out = (jnp.dot(hlo, w[:Dhin], preferred_element_type=F32)
               + jnp.dot(hhi, w[Dhin:], preferred_element_type=F32))
        out = out + b_ref[...]
        o_ref[...] = jnp.where(out > 0, out, 0.2 * out)

    bi = pl.BlockSpec((br, Dhin), lambda i: (i, 0))
    bc = pl.BlockSpec((br, 1), lambda i: (i, 0))
    return pl.pallas_call(
        body,
        grid=(N // br,),
        in_specs=[bi, bi,
                  pl.BlockSpec((DI, DO), lambda i: (0, 0)),
                  pl.BlockSpec((1, DO), lambda i: (0, 0)),
                  bc, bc],
        out_specs=pl.BlockSpec((br, DO), lambda i: (i, 0)),
        out_shape=jax.ShapeDtypeStruct((N, DO), F32),
    )(slo, shi, W, b.reshape(1, DO), p0, p1)


def kernel(x, edge_index, edge_attr, W1, b1, W2, b2, W3, b3, W_fc3, b_fc3):
    N, DI = x.shape
    E = edge_index.shape[1]
    NP = ((N + 2047) // 2048) * 2048

    EP = ((E + NW * 256 - 1) // (NW * 256)) * (NW * 256)
    row = jnp.pad(edge_index[0], (0, EP - E))
    col = jnp.pad(edge_index[1], (0, EP - E))
    ea = jnp.pad(edge_attr, (0, EP - E))
    xp = jnp.pad(x, ((0, NP - N), (0, 0)))

    deg_part = _make_deg(EP, NP)(col, ea)
    p0 = deg_part[0, :, 0:1]
    p1 = deg_part[1, :, 0:1]

    xq_lo, xq_hi, i_lo, i_hi = _mm_first(xp, W1, b1, p0, p1)
    s_lo, s_hi = _make_agg(EP, NP, W1.shape[1] // 2)(
        row, col, ea, xq_lo, xq_hi, i_lo, i_hi)

    xq_lo, xq_hi, i_lo, i_hi = _mm_mid(s_lo, s_hi, W2, b2, p0, p1)
    s_lo, s_hi = _make_agg(EP, NP, W2.shape[1] // 2)(
        row, col, ea, xq_lo, xq_hi, i_lo, i_hi)

    xq_lo, xq_hi, i_lo, i_hi = _mm_mid(s_lo, s_hi, W3, b3, p0, p1)
    s_lo, s_hi = _make_agg(EP, NP, W3.shape[1] // 2)(
        row, col, ea, xq_lo, xq_hi, i_lo, i_hi)

    return _head(s_lo, s_hi, W_fc3, b_fc3, p0, p1)[:N]
